# Initial kernel scaffold; baseline (speedup 1.0000x reference)
import jax
import jax.numpy as jnp
from jax.experimental import pallas as pl


def kernel(x, table):
    raise NotImplementedError("write your pallas kernel here")


# speedup vs baseline: 294.9386x; 294.9386x over previous
import dataclasses
import functools

import jax
import jax.numpy as jnp
from jax.experimental import pallas as pl
from jax.experimental.pallas import tpu as pltpu
from jax.experimental.pallas import tpu_sc as plsc

N_TABLE = 2048
SCALE = 64.0
MAGIC = 1.5 * 2.0**23
LANES = 16

BLK_ROWS = 128


def _sc_compiler_params():
    cp = pltpu.CompilerParams()
    if "needs_layout_passes" in pltpu.CompilerParams.__dataclass_fields__:
        cp = dataclasses.replace(cp, needs_layout_passes=False)
    return cp


def kernel(x, table):
    orig_shape = x.shape
    flat = x.reshape(-1, 128)
    rows = flat.shape[0]
    mesh = plsc.VectorSubcoreMesh(core_axis_name="c", subcore_axis_name="s")

    @functools.partial(
        pl.kernel,
        out_type=jax.ShapeDtypeStruct(flat.shape, jnp.float32),
        mesh=mesh,
        scratch_types=[pltpu.VMEM((N_TABLE,), jnp.float32)],
        compiler_params=_sc_compiler_params(),
    )
    def _lut_kernel(x_hbm, t_hbm, o_hbm, table_v):
        pltpu.sync_copy(t_hbm, table_v)

        def body(in_v, out_v):
            @pl.loop(0, BLK_ROWS)
            def _(r):
                for c in range(0, 128, LANES):
                    v = in_v[r, pl.ds(c, LANES)]
                    biased = v * SCALE + MAGIC
                    idx = plsc.bitcast(biased, jnp.int32) & (N_TABLE - 1)
                    out_v[r, pl.ds(c, LANES)] = plsc.load_gather(table_v, [idx])

        pltpu.emit_pipeline(
            body,
            grid=(rows // BLK_ROWS,),
            in_specs=[pl.BlockSpec((BLK_ROWS, 128), lambda i: (i, 0))],
            out_specs=[pl.BlockSpec((BLK_ROWS, 128), lambda i: (i, 0))],
            core_axis_name=("c", "s"),
            dimension_semantics=(pltpu.PARALLEL,),
        )(x_hbm, o_hbm)

    return _lut_kernel(flat, table).reshape(orig_shape)


# speedup vs baseline: 810.3135x; 2.7474x over previous
import dataclasses
import functools

import jax
import jax.numpy as jnp
from jax.experimental import pallas as pl
from jax.experimental.pallas import tpu as pltpu
from jax.experimental.pallas import tpu_sc as plsc

N_TABLE = 2048
SCALE = 64.0
MAGIC = 1.5 * 2.0**23
LANES = 16

BLK_ROWS = 128


def _sc_compiler_params():
    cp = pltpu.CompilerParams()
    if "needs_layout_passes" in pltpu.CompilerParams.__dataclass_fields__:
        cp = dataclasses.replace(cp, needs_layout_passes=False)
    return cp


def kernel(x, table):
    orig_shape = x.shape
    flat = x.reshape(-1, 128)
    rows = flat.shape[0]
    mesh = plsc.VectorSubcoreMesh(core_axis_name="c", subcore_axis_name="s")

    @functools.partial(
        pl.kernel,
        out_type=jax.ShapeDtypeStruct(flat.shape, jnp.float32),
        mesh=mesh,
        scratch_types=[pltpu.VMEM((N_TABLE,), jnp.float32)],
        compiler_params=_sc_compiler_params(),
    )
    def _lut_kernel(x_hbm, t_hbm, o_hbm, table_v):
        pltpu.sync_copy(t_hbm, table_v)

        def body(in_v, out_v):
            @plsc.parallel_loop(0, BLK_ROWS, unroll=2)
            def _(r):
                for c in range(0, 128, LANES):
                    v = in_v[r, pl.ds(c, LANES)]
                    biased = v * SCALE + MAGIC
                    idx = plsc.bitcast(biased, jnp.int32) & (N_TABLE - 1)
                    out_v[r, pl.ds(c, LANES)] = plsc.load_gather(table_v, [idx])

        pltpu.emit_pipeline(
            body,
            grid=(rows // BLK_ROWS,),
            in_specs=[pl.BlockSpec((BLK_ROWS, 128), lambda i: (i, 0))],
            out_specs=[pl.BlockSpec((BLK_ROWS, 128), lambda i: (i, 0))],
            core_axis_name=("c", "s"),
            dimension_semantics=(pltpu.PARALLEL,),
        )(x_hbm, o_hbm)

    return _lut_kernel(flat, table).reshape(orig_shape)


# speedup vs baseline: 2017.4672x; 2.4897x over previous
import dataclasses
import functools

import jax
import jax.numpy as jnp
from jax.experimental import pallas as pl
from jax.experimental.pallas import tpu as pltpu
from jax.experimental.pallas import tpu_sc as plsc

N_TABLE = 2048
SCALE = 64.0
MAGIC = 1.5 * 2.0**23
LANES = 16

BLK_ROWS = 128


def _sc_compiler_params():
    cp = pltpu.CompilerParams()
    if "needs_layout_passes" in pltpu.CompilerParams.__dataclass_fields__:
        cp = dataclasses.replace(cp, needs_layout_passes=False)
    return cp


def kernel(x, table):
    b, m, n = x.shape
    blk_m = 8
    mesh = plsc.VectorSubcoreMesh(core_axis_name="c", subcore_axis_name="s")

    @functools.partial(
        pl.kernel,
        out_type=jax.ShapeDtypeStruct(x.shape, jnp.float32),
        mesh=mesh,
        scratch_types=[pltpu.VMEM((N_TABLE,), jnp.float32)],
        compiler_params=_sc_compiler_params(),
    )
    def _lut_kernel(x_hbm, t_hbm, o_hbm, table_v):
        pltpu.sync_copy(t_hbm, table_v)

        def body(in_v, out_v):
            @pl.loop(0, blk_m)
            def _(r):
                @plsc.parallel_loop(0, n, step=8 * LANES, unroll=2)
                def _(c):
                    for k in range(8):
                        sl = pl.ds(c + k * LANES, LANES)
                        v = in_v[0, r, sl]
                        biased = v * SCALE + MAGIC
                        idx = plsc.bitcast(biased, jnp.int32) & (N_TABLE - 1)
                        out_v[0, r, sl] = plsc.load_gather(table_v, [idx])

        pltpu.emit_pipeline(
            body,
            grid=(b, m // blk_m),
            in_specs=[pl.BlockSpec((1, blk_m, n), lambda i, j: (i, j, 0))],
            out_specs=[pl.BlockSpec((1, blk_m, n), lambda i, j: (i, j, 0))],
            core_axis_name=("c", "s"),
            dimension_semantics=(pltpu.PARALLEL, pltpu.PARALLEL),
        )(x_hbm, o_hbm)

    return _lut_kernel(x, table)


# speedup vs baseline: 2166.3588x; 1.0738x over previous
import dataclasses
import functools

import jax
import jax.numpy as jnp
from jax.experimental import pallas as pl
from jax.experimental.pallas import tpu as pltpu
from jax.experimental.pallas import tpu_sc as plsc

N_TABLE = 2048
SCALE = 64.0
MAGIC = 1.5 * 2.0**23
LANES = 16

BLK_ROWS = 128


def _sc_compiler_params():
    cp = pltpu.CompilerParams()
    if "needs_layout_passes" in pltpu.CompilerParams.__dataclass_fields__:
        cp = dataclasses.replace(cp, needs_layout_passes=False)
    return cp


def kernel(x, table):
    b, m, n = x.shape
    blk_m = 8
    mesh = plsc.VectorSubcoreMesh(core_axis_name="c", subcore_axis_name="s")

    @functools.partial(
        pl.kernel,
        out_type=jax.ShapeDtypeStruct(x.shape, jnp.float32),
        mesh=mesh,
        scratch_types=[pltpu.VMEM((N_TABLE,), jnp.float32)],
        compiler_params=_sc_compiler_params(),
    )
    def _lut_kernel(x_hbm, t_hbm, o_hbm, table_v):
        pltpu.sync_copy(t_hbm, table_v)

        def body(in_v, out_v):
            @pl.loop(0, blk_m)
            def _(r):
                @plsc.parallel_loop(0, n, step=8 * LANES, unroll=4)
                def _(c):
                    for k in range(8):
                        sl = pl.ds(c + k * LANES, LANES)
                        v = in_v[0, r, sl]
                        biased = v * SCALE + MAGIC
                        idx = plsc.bitcast(biased, jnp.int32) & (N_TABLE - 1)
                        out_v[0, r, sl] = plsc.load_gather(table_v, [idx])

        pltpu.emit_pipeline(
            body,
            grid=(b, m // blk_m),
            in_specs=[pl.BlockSpec((1, blk_m, n), lambda i, j: (i, j, 0))],
            out_specs=[pl.BlockSpec((1, blk_m, n), lambda i, j: (i, j, 0))],
            core_axis_name=("c", "s"),
            dimension_semantics=(pltpu.PARALLEL, pltpu.PARALLEL),
        )(x_hbm, o_hbm)

    return _lut_kernel(x, table)


# speedup vs baseline: 2409.3826x; 1.1122x over previous
import dataclasses
import functools

import jax
import jax.numpy as jnp
from jax.experimental import pallas as pl
from jax.experimental.pallas import tpu as pltpu
from jax.experimental.pallas import tpu_sc as plsc

N_TABLE = 2048
SCALE = 64.0
MAGIC = 1.5 * 2.0**23
LANES = 16

BLK_ROWS = 128


def _sc_compiler_params():
    cp = pltpu.CompilerParams()
    if "needs_layout_passes" in pltpu.CompilerParams.__dataclass_fields__:
        cp = dataclasses.replace(cp, needs_layout_passes=False)
    return cp


def kernel(x, table):
    b, m, n = x.shape
    blk_m = 8
    mesh = plsc.VectorSubcoreMesh(core_axis_name="c", subcore_axis_name="s")

    @functools.partial(
        pl.kernel,
        out_type=jax.ShapeDtypeStruct(x.shape, jnp.float32),
        mesh=mesh,
        scratch_types=[pltpu.VMEM((N_TABLE,), jnp.float32)],
        compiler_params=_sc_compiler_params(),
    )
    def _lut_kernel(x_hbm, t_hbm, o_hbm, table_v):
        pltpu.sync_copy(t_hbm, table_v)

        def body(in_v, out_v):
            @plsc.parallel_loop(0, blk_m * n, step=8 * LANES, unroll=4)
            def _(i):
                r = jax.lax.shift_right_logical(i, 11)
                base = jax.lax.bitwise_and(i, n - 1)
                for k in range(8):
                    sl = pl.ds(base + k * LANES, LANES)
                    v = in_v[0, r, sl]
                    biased = v * SCALE + MAGIC
                    idx = plsc.bitcast(biased, jnp.int32) & (N_TABLE - 1)
                    out_v[0, r, sl] = plsc.load_gather(table_v, [idx])

        pltpu.emit_pipeline(
            body,
            grid=(b, m // blk_m),
            in_specs=[pl.BlockSpec((1, blk_m, n), lambda i, j: (i, j, 0))],
            out_specs=[pl.BlockSpec((1, blk_m, n), lambda i, j: (i, j, 0))],
            core_axis_name=("c", "s"),
            dimension_semantics=(pltpu.PARALLEL, pltpu.PARALLEL),
        )(x_hbm, o_hbm)

    return _lut_kernel(x, table)


# speedup vs baseline: 2450.0812x; 1.0169x over previous
import dataclasses
import functools

import jax
import jax.numpy as jnp
from jax.experimental import pallas as pl
from jax.experimental.pallas import tpu as pltpu
from jax.experimental.pallas import tpu_sc as plsc

N_TABLE = 2048
SCALE = 64.0
MAGIC = 1.5 * 2.0**23
LANES = 16

BLK_ROWS = 128


def _sc_compiler_params():
    cp = pltpu.CompilerParams()
    if "needs_layout_passes" in pltpu.CompilerParams.__dataclass_fields__:
        cp = dataclasses.replace(cp, needs_layout_passes=False)
    return cp


def kernel(x, table):
    b, m, n = x.shape
    blk_m = 8
    mesh = plsc.VectorSubcoreMesh(core_axis_name="c", subcore_axis_name="s")

    @functools.partial(
        pl.kernel,
        out_type=jax.ShapeDtypeStruct(x.shape, jnp.float32),
        mesh=mesh,
        scratch_types=[pltpu.VMEM((N_TABLE,), jnp.float32)],
        compiler_params=_sc_compiler_params(),
    )
    def _lut_kernel(x_hbm, t_hbm, o_hbm, table_v):
        pltpu.sync_copy(t_hbm, table_v)

        def body(in_v, out_v):
            @plsc.parallel_loop(0, blk_m * n, step=8 * LANES, unroll=8)
            def _(i):
                r = jax.lax.shift_right_logical(i, 11)
                base = jax.lax.bitwise_and(i, n - 1)
                for k in range(8):
                    sl = pl.ds(base + k * LANES, LANES)
                    v = in_v[0, r, sl]
                    biased = v * SCALE + MAGIC
                    idx = plsc.bitcast(biased, jnp.int32) & (N_TABLE - 1)
                    out_v[0, r, sl] = plsc.load_gather(table_v, [idx])

        pltpu.emit_pipeline(
            body,
            grid=(b, m // blk_m),
            in_specs=[pl.BlockSpec((1, blk_m, n), lambda i, j: (i, j, 0))],
            out_specs=[pl.BlockSpec((1, blk_m, n), lambda i, j: (i, j, 0))],
            core_axis_name=("c", "s"),
            dimension_semantics=(pltpu.PARALLEL, pltpu.PARALLEL),
        )(x_hbm, o_hbm)

    return _lut_kernel(x, table)
